# Initial kernel scaffold; baseline (speedup 1.0000x reference)
import jax
import jax.numpy as jnp
from jax.experimental import pallas as pl


def kernel(token_ids, table):
    raise NotImplementedError("write your pallas kernel here")


# speedup vs baseline: 1.4390x; 1.4390x over previous
import functools

import jax
import jax.numpy as jnp
from jax import lax
from jax.experimental import pallas as pl
from jax.experimental.pallas import tpu as pltpu
from jax.experimental.pallas import tpu_sc as plsc

EMBED_DIM = 768

_info = plsc.get_sparse_core_info()
NC, NS = _info.num_cores, _info.num_subcores
NW = NC * NS

CHUNK = 64


def _body(ids_hbm, table_hbm, out_hbm, idx_v, rows_v, sem):
    n_chunks = ids_hbm.shape[1]
    wid = lax.axis_index("s") * NC + lax.axis_index("c")
    base = wid * (n_chunks * CHUNK)
    pltpu.sync_copy(ids_hbm.at[wid], idx_v)

    def chunk(j, carry):
        pltpu.async_copy(table_hbm.at[idx_v.at[j]], rows_v, sem).wait()
        pltpu.sync_copy(rows_v, out_hbm.at[pl.ds(base + j * CHUNK, CHUNK)])
        return carry

    lax.fori_loop(0, n_chunks, chunk, 0)


def kernel(token_ids, table):
    b, s = token_ids.shape
    dim = table.shape[1]
    total = b * s
    assert total % (NW * CHUNK) == 0
    n_chunks = total // (NW * CHUNK)
    ids3 = token_ids.astype(jnp.int32).reshape(NW, n_chunks, CHUNK)

    emb = pl.kernel(
        _body,
        out_type=jax.ShapeDtypeStruct((total, dim), jnp.float32),
        mesh=plsc.VectorSubcoreMesh(core_axis_name="c", subcore_axis_name="s"),
        scratch_types=[
            pltpu.VMEM((n_chunks, CHUNK), jnp.int32),
            pltpu.VMEM((CHUNK, dim), jnp.float32),
            pltpu.SemaphoreType.DMA,
        ],
    )
    out = emb(ids3, table)
    return out.reshape(b, s, dim)


# speedup vs baseline: 1.4485x; 1.0066x over previous
import jax
import jax.numpy as jnp
from jax import lax
from jax.experimental import pallas as pl
from jax.experimental.pallas import tpu as pltpu
from jax.experimental.pallas import tpu_sc as plsc

EMBED_DIM = 768

_info = plsc.get_sparse_core_info()
NC, NS = _info.num_cores, _info.num_subcores
NW = NC * NS

CHUNK = 64


def _body(ids_hbm, table_hbm, out_hbm, idx_v, rows0, rows1, gsem0,
          gsem1, ssem0, ssem1):
    n_chunks = ids_hbm.shape[1]
    s = lax.axis_index("s")
    wid = s * NC + lax.axis_index("c")
    base = wid * (n_chunks * CHUNK)
    rows = (rows0, rows1)
    gsem = (gsem0, gsem1)
    ssem = (ssem0, ssem1)

    pltpu.sync_copy(ids_hbm.at[wid], idx_v)

    def gather(j, b):
        pltpu.async_copy(table_hbm.at[idx_v.at[j]], rows[b], gsem[b]).wait()

    def scatter_start(j, b):
        pltpu.async_copy(rows[b], out_hbm.at[pl.ds(base + j * CHUNK, CHUNK)],
                         ssem[b])

    def scatter_wait(j, b):
        pltpu.make_async_copy(rows[b],
                              out_hbm.at[pl.ds(base + j * CHUNK, CHUNK)],
                              ssem[b]).wait()

    for b in range(2):
        gather(b, b)
        scatter_start(b, b)

    def pair(g, carry):
        for b in range(2):
            j = 2 * g + b
            scatter_wait(j, b)
            gather(j, b)
            scatter_start(j, b)
        return carry

    lax.fori_loop(1, n_chunks // 2, pair, 0)

    for b in range(2):
        scatter_wait(0, b)


def kernel(token_ids, table):
    b, s = token_ids.shape
    dim = table.shape[1]
    total = b * s
    assert total % (NW * CHUNK * 2) == 0
    n_chunks = total // (NW * CHUNK)
    ids3 = token_ids.astype(jnp.int32).reshape(NW, n_chunks, CHUNK)

    emb = pl.kernel(
        _body,
        out_type=jax.ShapeDtypeStruct((total, dim), jnp.float32),
        mesh=plsc.VectorSubcoreMesh(core_axis_name="c", subcore_axis_name="s"),
        scratch_types=[
            pltpu.VMEM((n_chunks, CHUNK), jnp.int32),
            pltpu.VMEM((CHUNK, dim), jnp.float32),
            pltpu.VMEM((CHUNK, dim), jnp.float32),
            pltpu.SemaphoreType.DMA,
            pltpu.SemaphoreType.DMA,
            pltpu.SemaphoreType.DMA,
            pltpu.SemaphoreType.DMA,
        ],
    )
    out = emb(ids3, table)
    return out.reshape(b, s, dim)
